# baseline (speedup 1.0000x reference)
import functools

import jax
import jax.numpy as jnp
from jax import lax
from jax.experimental import pallas as pl
from jax.experimental.pallas import tpu as pltpu
from jax.experimental.pallas import tpu_sc as plsc

NUM_CORES = 2
NUM_SUBCORES = 16
NUM_WORKERS = NUM_CORES * NUM_SUBCORES
CHUNK = 128


def _sc_scatter_sum(h, srcp, dstp, zeros_nf):
    n, f = h.shape
    nacc = zeros_nf.shape[0]
    cpw = srcp.shape[0] // NUM_WORKERS
    npass = 2
    cpp = cpw // npass

    mesh = plsc.VectorSubcoreMesh(
        core_axis_name="c", subcore_axis_name="s",
        num_cores=NUM_CORES, num_subcores=NUM_SUBCORES)

    @functools.partial(
        pl.kernel,
        out_type=jax.ShapeDtypeStruct((NUM_CORES, n, f), jnp.float32),
        mesh=mesh,
        scratch_types=[
            pltpu.VMEM((cpp, CHUNK), jnp.int32),
            pltpu.VMEM((cpp, CHUNK), jnp.int32),
            pltpu.VMEM((CHUNK, f), jnp.float32),
            pltpu.VMEM((CHUNK, f), jnp.float32),
            pltpu.VMEM_SHARED((nacc, f), jnp.float32),
            pltpu.SemaphoreType.DMA,
            pltpu.SemaphoreType.DMA,
        ],
        compiler_params=pltpu.CompilerParams(use_tc_tiling_on_sc=False),
    )
    def k(h_hbm, src_hbm, dst_hbm, z_hbm, out_hbm,
          sidx, didx, rows0, rows1, acc, gsem0, gsem1):
        cid = lax.axis_index("c")
        sid = lax.axis_index("s")
        wid = cid * NUM_SUBCORES + sid

        @pl.when(sid == 0)
        def _():
            pltpu.sync_copy(z_hbm, acc)
        pltpu.sync_copy(src_hbm.at[pl.ds(wid * cpw, cpp)], sidx)
        pltpu.sync_copy(dst_hbm.at[pl.ds(wid * cpw, cpp)], didx)
        plsc.subcore_barrier()

        def body(i, _):
            c0 = i * 2
            pltpu.make_async_copy(h_hbm.at[sidx.at[c0]], rows0, gsem0).wait()
            pltpu.async_copy(h_hbm.at[sidx.at[c0 + 1]], rows1, gsem1)
            pltpu.sync_copy(rows0, acc.at[didx.at[c0]], add=True)

            @pl.when(c0 + 2 < cpp)
            def _():
                pltpu.async_copy(h_hbm.at[sidx.at[c0 + 2]], rows0, gsem0)

            pltpu.make_async_copy(h_hbm.at[sidx.at[c0 + 1]], rows1, gsem1).wait()
            pltpu.sync_copy(rows1, acc.at[didx.at[c0 + 1]], add=True)
            return ()

        for p in range(npass):
            if p > 0:
                pltpu.sync_copy(src_hbm.at[pl.ds(wid * cpw + p * cpp, cpp)], sidx)
                pltpu.sync_copy(dst_hbm.at[pl.ds(wid * cpw + p * cpp, cpp)], didx)
            pltpu.async_copy(h_hbm.at[sidx.at[0]], rows0, gsem0)
            lax.fori_loop(0, cpp // 2, body, (), unroll=False)

        plsc.subcore_barrier()

        @pl.when(sid == 0)
        def _():
            pltpu.sync_copy(acc.at[pl.ds(0, n)], out_hbm.at[cid])

    return k(h, srcp, dstp, zeros_nf)


def _tc_layer(h, agg, w0, w1, b1, w2, b2, gamma, beta, b_graphs, n_per):
    n, f = h.shape
    hdim = w2.shape[1]

    def body(h_ref, a_ref, w0_ref, w1_ref, b1_ref, w2_ref, b2_ref,
             g_ref, be_ref, out_ref, pool_ref):
        a = a_ref[...]
        rst = h_ref[...] + a[0] + a[1]
        if w0 is not None:
            rst = rst * w0_ref[...]
        z = jnp.dot(rst, w1_ref[...], preferred_element_type=jnp.float32)
        z = jnp.maximum(z + b1_ref[...], 0.0)
        z = jnp.dot(z, w2_ref[...], preferred_element_type=jnp.float32)
        hh = jnp.maximum(z + b2_ref[...], 0.0)
        mean = jnp.mean(hh, axis=0, keepdims=True)
        var = jnp.mean((hh - mean) * (hh - mean), axis=0, keepdims=True)
        hh = (hh - mean) * lax.rsqrt(var + 1e-5) * g_ref[...] + be_ref[...]
        out_ref[...] = hh
        rows = lax.broadcasted_iota(jnp.int32, (b_graphs, n), 1) // n_per
        seg = lax.broadcasted_iota(jnp.int32, (b_graphs, n), 0)
        ind = jnp.where(rows == seg, 1.0, 0.0).astype(jnp.float32)
        pool_ref[...] = jnp.dot(ind, hh, preferred_element_type=jnp.float32)

    args = [h, agg]
    if w0 is None:
        w0_in = jnp.zeros((1, f), jnp.float32)
    else:
        w0_in = w0
    args += [w0_in, w1, b1.reshape(1, -1), w2, b2.reshape(1, -1),
             gamma.reshape(1, -1), beta.reshape(1, -1)]
    return pl.pallas_call(
        body,
        out_shape=[
            jax.ShapeDtypeStruct((n, hdim), jnp.float32),
            jax.ShapeDtypeStruct((b_graphs, hdim), jnp.float32),
        ],
    )(*args)


def _tc_combine(p1, p2, p3, n_per):
    b, hdim = p1.shape

    def body(p1_ref, p2_ref, p3_ref, xcat_ref, emb_ref):
        xcat_ref[:, 0:hdim] = p1_ref[...]
        xcat_ref[:, hdim:2 * hdim] = p2_ref[...]
        xcat_ref[:, 2 * hdim:3 * hdim] = p3_ref[...]
        emb_ref[:, 0:hdim] = p1_ref[...] * (1.0 / n_per)
        emb_ref[:, hdim:2 * hdim] = p2_ref[...] * (1.0 / n_per)
        emb_ref[:, 2 * hdim:3 * hdim] = p3_ref[...] * (1.0 / n_per)

    return pl.pallas_call(
        body,
        out_shape=[
            jax.ShapeDtypeStruct((b, 3 * hdim), jnp.float32),
            jax.ShapeDtypeStruct((b, 3 * hdim), jnp.float32),
        ],
    )(p1, p2, p3)


def kernel(x, w0, params, edge_index, graph_len, prompt_id, scalar):
    n, d = x.shape
    e = edge_index.shape[1]
    b_graphs = graph_len.shape[0]
    n_per = n // b_graphs
    hdim = params[0][2].shape[1]

    epw = -(-e // (NUM_WORKERS * 2 * CHUNK)) * (2 * CHUNK)
    ep = epw * NUM_WORKERS
    srcp = jnp.concatenate(
        [edge_index[0], jnp.zeros((ep - e,), jnp.int32)]).reshape(-1, CHUNK)
    dstp = jnp.concatenate(
        [edge_index[1], jnp.full((ep - e,), n, jnp.int32)]).reshape(-1, CHUNK)

    zeros_d = jnp.zeros((n + 8, d), jnp.float32)
    zeros_h = jnp.zeros((n + 8, hdim), jnp.float32)

    agg = _sc_scatter_sum(x, srcp, dstp, zeros_d)
    w1, b1, w2, b2, gamma, beta = params[0]
    h, p1 = _tc_layer(x, agg, w0, w1, b1, w2, b2, gamma, beta, b_graphs, n_per)

    w1, b1, w2, b2, gamma, beta = params[1]
    agg = _sc_scatter_sum(h, srcp, dstp, zeros_h)
    h, p2 = _tc_layer(h, agg, None, w1, b1, w2, b2, gamma, beta, b_graphs, n_per)

    w1, b1, w2, b2, gamma, beta = params[2]
    agg = _sc_scatter_sum(h, srcp, dstp, zeros_h)
    h, p3 = _tc_layer(h, agg, None, w1, b1, w2, b2, gamma, beta, b_graphs, n_per)

    xcat, emb = _tc_combine(p1, p2, p3, n_per)
    return (xcat, emb)


# speedup vs baseline: 1.0229x; 1.0229x over previous
import functools

import jax
import jax.numpy as jnp
from jax import lax
from jax.experimental import pallas as pl
from jax.experimental.pallas import tpu as pltpu
from jax.experimental.pallas import tpu_sc as plsc

NUM_CORES = 2
NUM_SUBCORES = 16
NUM_WORKERS = NUM_CORES * NUM_SUBCORES
CHUNK = 128


def _sc_scatter_sum(h, srcp, dstp, zeros_nf):
    n, f = h.shape
    nacc = zeros_nf.shape[0]
    cpw = srcp.shape[0] // NUM_WORKERS

    mesh = plsc.VectorSubcoreMesh(
        core_axis_name="c", subcore_axis_name="s",
        num_cores=NUM_CORES, num_subcores=NUM_SUBCORES)

    @functools.partial(
        pl.kernel,
        out_type=jax.ShapeDtypeStruct((NUM_CORES, n, f), jnp.float32),
        mesh=mesh,
        scratch_types=[
            pltpu.VMEM((CHUNK,), jnp.int32),
            pltpu.VMEM((CHUNK,), jnp.int32),
            pltpu.VMEM((CHUNK,), jnp.int32),
            pltpu.VMEM((CHUNK,), jnp.int32),
            pltpu.VMEM((CHUNK, f), jnp.float32),
            pltpu.VMEM((CHUNK, f), jnp.float32),
            pltpu.VMEM_SHARED((nacc, f), jnp.float32),
            pltpu.SemaphoreType.DMA,
            pltpu.SemaphoreType.DMA,
        ],
        compiler_params=pltpu.CompilerParams(use_tc_tiling_on_sc=False),
    )
    def k(h_hbm, src_hbm, dst_hbm, z_hbm, out_hbm,
          sidx0, sidx1, didx0, didx1, rows0, rows1, acc, gsem0, gsem1):
        cid = lax.axis_index("c")
        sid = lax.axis_index("s")
        wid = cid * NUM_SUBCORES + sid
        base = wid * cpw

        @pl.when(sid == 0)
        def _():
            pltpu.sync_copy(z_hbm, acc)
        plsc.subcore_barrier()

        def fire(c, sidx, didx, rows, gsem):
            pltpu.sync_copy(src_hbm.at[c], sidx)
            pltpu.sync_copy(dst_hbm.at[c], didx)
            pltpu.async_copy(h_hbm.at[sidx], rows, gsem)

        fire(base, sidx0, didx0, rows0, gsem0)

        def body(i, _):
            c0 = base + i * 2
            fire(c0 + 1, sidx1, didx1, rows1, gsem1)
            pltpu.make_async_copy(h_hbm.at[sidx0], rows0, gsem0).wait()
            pltpu.sync_copy(rows0, acc.at[didx0], add=True)

            @pl.when(i * 2 + 2 < cpw)
            def _():
                fire(c0 + 2, sidx0, didx0, rows0, gsem0)

            pltpu.make_async_copy(h_hbm.at[sidx1], rows1, gsem1).wait()
            pltpu.sync_copy(rows1, acc.at[didx1], add=True)
            return ()

        lax.fori_loop(0, cpw // 2, body, (), unroll=False)

        plsc.subcore_barrier()

        @pl.when(sid == 0)
        def _():
            pltpu.sync_copy(acc.at[pl.ds(0, n)], out_hbm.at[cid])

    return k(h, srcp, dstp, zeros_nf)


def _tc_layer(h, agg, w0, w1, b1, w2, b2, gamma, beta, b_graphs, n_per):
    n, f = h.shape
    hdim = w2.shape[1]

    def body(h_ref, a_ref, w0_ref, w1_ref, b1_ref, w2_ref, b2_ref,
             g_ref, be_ref, out_ref, pool_ref):
        a = a_ref[...]
        rst = h_ref[...] + a[0] + a[1]
        if w0 is not None:
            rst = rst * w0_ref[...]
        z = jnp.dot(rst, w1_ref[...], preferred_element_type=jnp.float32)
        z = jnp.maximum(z + b1_ref[...], 0.0)
        z = jnp.dot(z, w2_ref[...], preferred_element_type=jnp.float32)
        hh = jnp.maximum(z + b2_ref[...], 0.0)
        mean = jnp.mean(hh, axis=0, keepdims=True)
        var = jnp.mean((hh - mean) * (hh - mean), axis=0, keepdims=True)
        hh = (hh - mean) * lax.rsqrt(var + 1e-5) * g_ref[...] + be_ref[...]
        out_ref[...] = hh
        rows = lax.broadcasted_iota(jnp.int32, (b_graphs, n), 1) // n_per
        seg = lax.broadcasted_iota(jnp.int32, (b_graphs, n), 0)
        ind = jnp.where(rows == seg, 1.0, 0.0).astype(jnp.float32)
        pool_ref[...] = jnp.dot(ind, hh, preferred_element_type=jnp.float32)

    args = [h, agg]
    if w0 is None:
        w0_in = jnp.zeros((1, f), jnp.float32)
    else:
        w0_in = w0
    args += [w0_in, w1, b1.reshape(1, -1), w2, b2.reshape(1, -1),
             gamma.reshape(1, -1), beta.reshape(1, -1)]
    return pl.pallas_call(
        body,
        out_shape=[
            jax.ShapeDtypeStruct((n, hdim), jnp.float32),
            jax.ShapeDtypeStruct((b_graphs, hdim), jnp.float32),
        ],
    )(*args)


def _tc_combine(p1, p2, p3, n_per):
    b, hdim = p1.shape

    def body(p1_ref, p2_ref, p3_ref, xcat_ref, emb_ref):
        xcat_ref[:, 0:hdim] = p1_ref[...]
        xcat_ref[:, hdim:2 * hdim] = p2_ref[...]
        xcat_ref[:, 2 * hdim:3 * hdim] = p3_ref[...]
        emb_ref[:, 0:hdim] = p1_ref[...] * (1.0 / n_per)
        emb_ref[:, hdim:2 * hdim] = p2_ref[...] * (1.0 / n_per)
        emb_ref[:, 2 * hdim:3 * hdim] = p3_ref[...] * (1.0 / n_per)

    return pl.pallas_call(
        body,
        out_shape=[
            jax.ShapeDtypeStruct((b, 3 * hdim), jnp.float32),
            jax.ShapeDtypeStruct((b, 3 * hdim), jnp.float32),
        ],
    )(p1, p2, p3)


def kernel(x, w0, params, edge_index, graph_len, prompt_id, scalar):
    n, d = x.shape
    e = edge_index.shape[1]
    b_graphs = graph_len.shape[0]
    n_per = n // b_graphs
    hdim = params[0][2].shape[1]

    epw = -(-e // (NUM_WORKERS * 2 * CHUNK)) * (2 * CHUNK)
    ep = epw * NUM_WORKERS
    srcp = jnp.concatenate(
        [edge_index[0], jnp.zeros((ep - e,), jnp.int32)]).reshape(-1, CHUNK)
    dstp = jnp.concatenate(
        [edge_index[1], jnp.full((ep - e,), n, jnp.int32)]).reshape(-1, CHUNK)

    zeros_d = jnp.zeros((n + 8, d), jnp.float32)
    zeros_h = jnp.zeros((n + 8, hdim), jnp.float32)

    agg = _sc_scatter_sum(x, srcp, dstp, zeros_d)
    w1, b1, w2, b2, gamma, beta = params[0]
    h, p1 = _tc_layer(x, agg, w0, w1, b1, w2, b2, gamma, beta, b_graphs, n_per)

    w1, b1, w2, b2, gamma, beta = params[1]
    agg = _sc_scatter_sum(h, srcp, dstp, zeros_h)
    h, p2 = _tc_layer(h, agg, None, w1, b1, w2, b2, gamma, beta, b_graphs, n_per)

    w1, b1, w2, b2, gamma, beta = params[2]
    agg = _sc_scatter_sum(h, srcp, dstp, zeros_h)
    h, p3 = _tc_layer(h, agg, None, w1, b1, w2, b2, gamma, beta, b_graphs, n_per)

    xcat, emb = _tc_combine(p1, p2, p3, n_per)
    return (xcat, emb)


# speedup vs baseline: 2.2656x; 2.2149x over previous
import functools

import jax
import jax.numpy as jnp
from jax import lax
from jax.experimental import pallas as pl
from jax.experimental.pallas import tpu as pltpu
from jax.experimental.pallas import tpu_sc as plsc

NUM_CORES = 2
NUM_SUBCORES = 16
NUM_WORKERS = NUM_CORES * NUM_SUBCORES
CHUNK = 128


def _sc_scatter_sum(h, srcp, dstp, zeros_nf):
    n, f = h.shape
    nacc = zeros_nf.shape[0]
    cpw = srcp.shape[0] // NUM_WORKERS

    mesh = plsc.VectorSubcoreMesh(
        core_axis_name="c", subcore_axis_name="s",
        num_cores=NUM_CORES, num_subcores=NUM_SUBCORES)

    @functools.partial(
        pl.kernel,
        out_type=jax.ShapeDtypeStruct((NUM_CORES, n, f), jnp.float32),
        mesh=mesh,
        scratch_types=[
            pltpu.VMEM((CHUNK,), jnp.int32),
            pltpu.VMEM((CHUNK,), jnp.int32),
            pltpu.VMEM((CHUNK,), jnp.int32),
            pltpu.VMEM((CHUNK,), jnp.int32),
            pltpu.VMEM((CHUNK, f), jnp.float32),
            pltpu.VMEM((CHUNK, f), jnp.float32),
            pltpu.VMEM_SHARED((nacc, f), jnp.float32),
            pltpu.SemaphoreType.DMA,
            pltpu.SemaphoreType.DMA,
        ],
        compiler_params=pltpu.CompilerParams(use_tc_tiling_on_sc=False),
    )
    def k(h_hbm, src_hbm, dst_hbm, z_hbm, out_hbm,
          sidx0, sidx1, didx0, didx1, rows0, rows1, acc, gsem0, gsem1):
        cid = lax.axis_index("c")
        sid = lax.axis_index("s")
        wid = cid * NUM_SUBCORES + sid
        base = wid * cpw

        @pl.when(sid == 0)
        def _():
            pltpu.sync_copy(z_hbm, acc)
        plsc.subcore_barrier()

        def fire(c, sidx, didx, rows, gsem):
            pltpu.sync_copy(src_hbm.at[c], sidx)
            pltpu.sync_copy(dst_hbm.at[c], didx)
            pltpu.async_copy(h_hbm.at[sidx], rows, gsem)

        fire(base, sidx0, didx0, rows0, gsem0)

        def body(i, _):
            c0 = base + i * 2
            fire(c0 + 1, sidx1, didx1, rows1, gsem1)
            pltpu.make_async_copy(h_hbm.at[sidx0], rows0, gsem0).wait()
            pltpu.sync_copy(rows0, acc.at[didx0], add=True)

            @pl.when(i * 2 + 2 < cpw)
            def _():
                fire(c0 + 2, sidx0, didx0, rows0, gsem0)

            pltpu.make_async_copy(h_hbm.at[sidx1], rows1, gsem1).wait()
            pltpu.sync_copy(rows1, acc.at[didx1], add=True)
            return ()

        lax.fori_loop(0, cpw // 2, body, (), unroll=False)

        plsc.subcore_barrier()

        @pl.when(sid == 0)
        def _():
            pltpu.sync_copy(acc.at[pl.ds(0, n)], out_hbm.at[cid])

    return k(h, srcp, dstp, zeros_nf)


def _tc_layer(h, agg, w0, w1, b1, w2, b2, gamma, beta, b_graphs, n_per):
    n, f = h.shape
    hdim = w2.shape[1]

    def body(h_ref, a_ref, w0_ref, w1_ref, b1_ref, w2_ref, b2_ref,
             g_ref, be_ref, out_ref, pool_ref):
        a = a_ref[...]
        rst = h_ref[...] + a[0] + a[1]
        if w0 is not None:
            rst = rst * w0_ref[...]
        z = jnp.dot(rst, w1_ref[...], preferred_element_type=jnp.float32)
        z = jnp.maximum(z + b1_ref[...], 0.0)
        z = jnp.dot(z, w2_ref[...], preferred_element_type=jnp.float32)
        hh = jnp.maximum(z + b2_ref[...], 0.0)
        mean = jnp.mean(hh, axis=0, keepdims=True)
        var = jnp.mean((hh - mean) * (hh - mean), axis=0, keepdims=True)
        hh = (hh - mean) * lax.rsqrt(var + 1e-5) * g_ref[...] + be_ref[...]
        out_ref[...] = hh
        rows = lax.broadcasted_iota(jnp.int32, (b_graphs, n), 1) // n_per
        seg = lax.broadcasted_iota(jnp.int32, (b_graphs, n), 0)
        ind = jnp.where(rows == seg, 1.0, 0.0).astype(jnp.float32)
        pool_ref[...] = jnp.dot(ind, hh, preferred_element_type=jnp.float32)

    args = [h, agg]
    if w0 is None:
        w0_in = jnp.zeros((1, f), jnp.float32)
    else:
        w0_in = w0
    args += [w0_in, w1, b1.reshape(1, -1), w2, b2.reshape(1, -1),
             gamma.reshape(1, -1), beta.reshape(1, -1)]
    return pl.pallas_call(
        body,
        out_shape=[
            jax.ShapeDtypeStruct((n, hdim), jnp.float32),
            jax.ShapeDtypeStruct((b_graphs, hdim), jnp.float32),
        ],
    )(*args)


def _tc_combine(p1, p2, p3, n_per):
    b, hdim = p1.shape

    def body(p1_ref, p2_ref, p3_ref, xcat_ref, emb_ref):
        xcat_ref[:, 0:hdim] = p1_ref[...]
        xcat_ref[:, hdim:2 * hdim] = p2_ref[...]
        xcat_ref[:, 2 * hdim:3 * hdim] = p3_ref[...]
        emb_ref[:, 0:hdim] = p1_ref[...] * (1.0 / n_per)
        emb_ref[:, hdim:2 * hdim] = p2_ref[...] * (1.0 / n_per)
        emb_ref[:, 2 * hdim:3 * hdim] = p3_ref[...] * (1.0 / n_per)

    return pl.pallas_call(
        body,
        out_shape=[
            jax.ShapeDtypeStruct((b, 3 * hdim), jnp.float32),
            jax.ShapeDtypeStruct((b, 3 * hdim), jnp.float32),
        ],
    )(p1, p2, p3)


def kernel(x, w0, params, edge_index, graph_len, prompt_id, scalar):
    n, d = x.shape
    e = edge_index.shape[1]
    b_graphs = graph_len.shape[0]
    n_per = n // b_graphs
    hdim = params[0][2].shape[1]

    SPREAD = 128
    epw = -(-e // (NUM_WORKERS * 2 * CHUNK)) * (2 * CHUNK)
    ep = epw * NUM_WORKERS
    pad_ar = jnp.arange(ep - e, dtype=jnp.int32)
    srcp = jnp.concatenate(
        [edge_index[0], pad_ar % n]).reshape(-1, CHUNK)
    dstp = jnp.concatenate(
        [edge_index[1], n + pad_ar % SPREAD]).reshape(-1, CHUNK)

    zeros_d = jnp.zeros((n + SPREAD, d), jnp.float32)
    zeros_h = jnp.zeros((n + SPREAD, hdim), jnp.float32)

    agg = _sc_scatter_sum(x, srcp, dstp, zeros_d)
    w1, b1, w2, b2, gamma, beta = params[0]
    h, p1 = _tc_layer(x, agg, w0, w1, b1, w2, b2, gamma, beta, b_graphs, n_per)

    w1, b1, w2, b2, gamma, beta = params[1]
    agg = _sc_scatter_sum(h, srcp, dstp, zeros_h)
    h, p2 = _tc_layer(h, agg, None, w1, b1, w2, b2, gamma, beta, b_graphs, n_per)

    w1, b1, w2, b2, gamma, beta = params[2]
    agg = _sc_scatter_sum(h, srcp, dstp, zeros_h)
    h, p3 = _tc_layer(h, agg, None, w1, b1, w2, b2, gamma, beta, b_graphs, n_per)

    xcat, emb = _tc_combine(p1, p2, p3, n_per)
    return (xcat, emb)


# speedup vs baseline: 2.8169x; 1.2434x over previous
import functools

import jax
import jax.numpy as jnp
from jax import lax
from jax.experimental import pallas as pl
from jax.experimental.pallas import tpu as pltpu
from jax.experimental.pallas import tpu_sc as plsc

NUM_CORES = 2
NUM_SUBCORES = 16
NUM_WORKERS = NUM_CORES * NUM_SUBCORES
CHUNK = 128
NBUF = 3


def _sc_scatter_sum(h, spd, zeros_nf):
    n, f = h.shape
    nacc = zeros_nf.shape[0]
    cpw = spd.shape[0] // NUM_WORKERS

    mesh = plsc.VectorSubcoreMesh(
        core_axis_name="c", subcore_axis_name="s",
        num_cores=NUM_CORES, num_subcores=NUM_SUBCORES)

    @functools.partial(
        pl.kernel,
        out_type=jax.ShapeDtypeStruct((NUM_CORES, n, f), jnp.float32),
        mesh=mesh,
        scratch_types=[
            [pltpu.VMEM((2, CHUNK), jnp.int32) for _ in range(NBUF)],
            [pltpu.VMEM((CHUNK, f), jnp.float32) for _ in range(NBUF)],
            pltpu.VMEM_SHARED((nacc, f), jnp.float32),
            [pltpu.SemaphoreType.DMA for _ in range(NBUF)],
        ],
        compiler_params=pltpu.CompilerParams(use_tc_tiling_on_sc=False),
    )
    def k(h_hbm, spd_hbm, z_hbm, out_hbm, idx, rows, acc, gsem):
        cid = lax.axis_index("c")
        sid = lax.axis_index("s")
        wid = cid * NUM_SUBCORES + sid
        base = wid * cpw

        @pl.when(sid == 0)
        def _():
            pltpu.sync_copy(z_hbm, acc)
        plsc.subcore_barrier()

        def fire(c, j):
            pltpu.sync_copy(spd_hbm.at[c], idx[j])
            pltpu.async_copy(h_hbm.at[idx[j].at[0]], rows[j], gsem[j])

        for j in range(NBUF):
            fire(base + j, j)

        def body(g, _):
            c0 = base + g * NBUF
            for j in range(NBUF):
                pltpu.make_async_copy(
                    h_hbm.at[idx[j].at[0]], rows[j], gsem[j]).wait()
                pltpu.sync_copy(rows[j], acc.at[idx[j].at[1]], add=True)

                @pl.when(g * NBUF + j + NBUF < cpw)
                def _():
                    fire(c0 + j + NBUF, j)
            return ()

        lax.fori_loop(0, cpw // NBUF, body, (), unroll=False)

        plsc.subcore_barrier()

        @pl.when(sid == 0)
        def _():
            pltpu.sync_copy(acc.at[pl.ds(0, n)], out_hbm.at[cid])

    return k(h, spd, zeros_nf)


def _tc_layer(h, agg, w0, w1, b1, w2, b2, gamma, beta, b_graphs, n_per):
    n, f = h.shape
    hdim = w2.shape[1]

    def body(h_ref, a_ref, w0_ref, w1_ref, b1_ref, w2_ref, b2_ref,
             g_ref, be_ref, out_ref, pool_ref):
        a = a_ref[...]
        rst = h_ref[...] + a[0] + a[1]
        if w0 is not None:
            rst = rst * w0_ref[...]
        z = jnp.dot(rst, w1_ref[...], preferred_element_type=jnp.float32)
        z = jnp.maximum(z + b1_ref[...], 0.0)
        z = jnp.dot(z, w2_ref[...], preferred_element_type=jnp.float32)
        hh = jnp.maximum(z + b2_ref[...], 0.0)
        mean = jnp.mean(hh, axis=0, keepdims=True)
        var = jnp.mean((hh - mean) * (hh - mean), axis=0, keepdims=True)
        hh = (hh - mean) * lax.rsqrt(var + 1e-5) * g_ref[...] + be_ref[...]
        out_ref[...] = hh
        rows = lax.broadcasted_iota(jnp.int32, (b_graphs, n), 1) // n_per
        seg = lax.broadcasted_iota(jnp.int32, (b_graphs, n), 0)
        ind = jnp.where(rows == seg, 1.0, 0.0).astype(jnp.float32)
        pool_ref[...] = jnp.dot(ind, hh, preferred_element_type=jnp.float32)

    args = [h, agg]
    if w0 is None:
        w0_in = jnp.zeros((1, f), jnp.float32)
    else:
        w0_in = w0
    args += [w0_in, w1, b1.reshape(1, -1), w2, b2.reshape(1, -1),
             gamma.reshape(1, -1), beta.reshape(1, -1)]
    return pl.pallas_call(
        body,
        out_shape=[
            jax.ShapeDtypeStruct((n, hdim), jnp.float32),
            jax.ShapeDtypeStruct((b_graphs, hdim), jnp.float32),
        ],
    )(*args)


def _tc_combine(p1, p2, p3, n_per):
    b, hdim = p1.shape

    def body(p1_ref, p2_ref, p3_ref, xcat_ref, emb_ref):
        xcat_ref[:, 0:hdim] = p1_ref[...]
        xcat_ref[:, hdim:2 * hdim] = p2_ref[...]
        xcat_ref[:, 2 * hdim:3 * hdim] = p3_ref[...]
        emb_ref[:, 0:hdim] = p1_ref[...] * (1.0 / n_per)
        emb_ref[:, hdim:2 * hdim] = p2_ref[...] * (1.0 / n_per)
        emb_ref[:, 2 * hdim:3 * hdim] = p3_ref[...] * (1.0 / n_per)

    return pl.pallas_call(
        body,
        out_shape=[
            jax.ShapeDtypeStruct((b, 3 * hdim), jnp.float32),
            jax.ShapeDtypeStruct((b, 3 * hdim), jnp.float32),
        ],
    )(p1, p2, p3)


def kernel(x, w0, params, edge_index, graph_len, prompt_id, scalar):
    n, d = x.shape
    e = edge_index.shape[1]
    b_graphs = graph_len.shape[0]
    n_per = n // b_graphs
    hdim = params[0][2].shape[1]

    SPREAD = 64
    grp = NUM_WORKERS * NBUF * CHUNK
    ep = -(-e // grp) * grp
    pad_ar = jnp.arange(ep - e, dtype=jnp.int32)
    srcp = jnp.concatenate([edge_index[0], pad_ar % n]).reshape(-1, CHUNK)
    dstp = jnp.concatenate([edge_index[1], n + pad_ar % SPREAD]).reshape(-1, CHUNK)
    spd = jnp.stack([srcp, dstp], axis=1)

    zeros_d = jnp.zeros((n + SPREAD, d), jnp.float32)
    zeros_h = jnp.zeros((n + SPREAD, hdim), jnp.float32)

    agg = _sc_scatter_sum(x, spd, zeros_d)
    w1, b1, w2, b2, gamma, beta = params[0]
    h, p1 = _tc_layer(x, agg, w0, w1, b1, w2, b2, gamma, beta, b_graphs, n_per)

    w1, b1, w2, b2, gamma, beta = params[1]
    agg = _sc_scatter_sum(h, spd, zeros_h)
    h, p2 = _tc_layer(h, agg, None, w1, b1, w2, b2, gamma, beta, b_graphs, n_per)

    w1, b1, w2, b2, gamma, beta = params[2]
    agg = _sc_scatter_sum(h, spd, zeros_h)
    h, p3 = _tc_layer(h, agg, None, w1, b1, w2, b2, gamma, beta, b_graphs, n_per)

    xcat, emb = _tc_combine(p1, p2, p3, n_per)
    return (xcat, emb)


# speedup vs baseline: 2.8231x; 1.0022x over previous
import functools

import jax
import jax.numpy as jnp
from jax import lax
from jax.experimental import pallas as pl
from jax.experimental.pallas import tpu as pltpu
from jax.experimental.pallas import tpu_sc as plsc

NUM_CORES = 2
NUM_SUBCORES = 16
NUM_WORKERS = NUM_CORES * NUM_SUBCORES
CHUNK = 128
NBUF = 3


def _sc_scatter_sum(h, spd, zeros_nf):
    n, f = h.shape
    nacc = zeros_nf.shape[0]
    cpw = spd.shape[0] // NUM_WORKERS

    mesh = plsc.VectorSubcoreMesh(
        core_axis_name="c", subcore_axis_name="s",
        num_cores=NUM_CORES, num_subcores=NUM_SUBCORES)

    @functools.partial(
        pl.kernel,
        out_type=jax.ShapeDtypeStruct((NUM_CORES, n, f), jnp.float32),
        mesh=mesh,
        scratch_types=[
            [pltpu.VMEM((2, CHUNK), jnp.int32) for _ in range(NBUF)],
            [pltpu.VMEM((CHUNK, f), jnp.float32) for _ in range(NBUF)],
            pltpu.VMEM_SHARED((nacc, f), jnp.float32),
            [pltpu.SemaphoreType.DMA for _ in range(NBUF)],
        ],
        compiler_params=pltpu.CompilerParams(use_tc_tiling_on_sc=False),
    )
    def k(h_hbm, spd_hbm, z_hbm, out_hbm, idx, rows, acc, gsem):
        cid = lax.axis_index("c")
        sid = lax.axis_index("s")
        wid = cid * NUM_SUBCORES + sid
        base = wid * cpw

        def fire(c, j):
            pltpu.sync_copy(spd_hbm.at[c], idx[j])
            pltpu.async_copy(h_hbm.at[idx[j].at[0]], rows[j], gsem[j])

        for j in range(NBUF):
            fire(base + j, j)

        rz = nacc // NUM_SUBCORES
        pltpu.sync_copy(z_hbm.at[pl.ds(sid * rz, rz)],
                        acc.at[pl.ds(sid * rz, rz)])
        plsc.subcore_barrier()

        def body(g, _):
            c0 = base + g * NBUF
            for j in range(NBUF):
                pltpu.make_async_copy(
                    h_hbm.at[idx[j].at[0]], rows[j], gsem[j]).wait()
                pltpu.sync_copy(rows[j], acc.at[idx[j].at[1]], add=True)

                @pl.when(g * NBUF + j + NBUF < cpw)
                def _():
                    fire(c0 + j + NBUF, j)
            return ()

        lax.fori_loop(0, cpw // NBUF, body, (), unroll=False)

        plsc.subcore_barrier()

        rw = n // NUM_SUBCORES
        pltpu.sync_copy(acc.at[pl.ds(sid * rw, rw)],
                        out_hbm.at[cid, pl.ds(sid * rw, rw)])

    return k(h, spd, zeros_nf)


def _tc_layer(h, agg, w0, w1, b1, w2, b2, gamma, beta, b_graphs, n_per):
    n, f = h.shape
    hdim = w2.shape[1]

    def body(h_ref, a_ref, w0_ref, w1_ref, b1_ref, w2_ref, b2_ref,
             g_ref, be_ref, out_ref, pool_ref):
        a = a_ref[...]
        rst = h_ref[...] + a[0] + a[1]
        if w0 is not None:
            rst = rst * w0_ref[...]
        z = jnp.dot(rst, w1_ref[...], preferred_element_type=jnp.float32)
        z = jnp.maximum(z + b1_ref[...], 0.0)
        z = jnp.dot(z, w2_ref[...], preferred_element_type=jnp.float32)
        hh = jnp.maximum(z + b2_ref[...], 0.0)
        mean = jnp.mean(hh, axis=0, keepdims=True)
        var = jnp.mean((hh - mean) * (hh - mean), axis=0, keepdims=True)
        hh = (hh - mean) * lax.rsqrt(var + 1e-5) * g_ref[...] + be_ref[...]
        out_ref[...] = hh
        rows = lax.broadcasted_iota(jnp.int32, (b_graphs, n), 1) // n_per
        seg = lax.broadcasted_iota(jnp.int32, (b_graphs, n), 0)
        ind = jnp.where(rows == seg, 1.0, 0.0).astype(jnp.float32)
        pool_ref[...] = jnp.dot(ind, hh, preferred_element_type=jnp.float32)

    args = [h, agg]
    if w0 is None:
        w0_in = jnp.zeros((1, f), jnp.float32)
    else:
        w0_in = w0
    args += [w0_in, w1, b1.reshape(1, -1), w2, b2.reshape(1, -1),
             gamma.reshape(1, -1), beta.reshape(1, -1)]
    return pl.pallas_call(
        body,
        out_shape=[
            jax.ShapeDtypeStruct((n, hdim), jnp.float32),
            jax.ShapeDtypeStruct((b_graphs, hdim), jnp.float32),
        ],
    )(*args)


def _tc_combine(p1, p2, p3, n_per):
    b, hdim = p1.shape

    def body(p1_ref, p2_ref, p3_ref, xcat_ref, emb_ref):
        xcat_ref[:, 0:hdim] = p1_ref[...]
        xcat_ref[:, hdim:2 * hdim] = p2_ref[...]
        xcat_ref[:, 2 * hdim:3 * hdim] = p3_ref[...]
        emb_ref[:, 0:hdim] = p1_ref[...] * (1.0 / n_per)
        emb_ref[:, hdim:2 * hdim] = p2_ref[...] * (1.0 / n_per)
        emb_ref[:, 2 * hdim:3 * hdim] = p3_ref[...] * (1.0 / n_per)

    return pl.pallas_call(
        body,
        out_shape=[
            jax.ShapeDtypeStruct((b, 3 * hdim), jnp.float32),
            jax.ShapeDtypeStruct((b, 3 * hdim), jnp.float32),
        ],
    )(p1, p2, p3)


def kernel(x, w0, params, edge_index, graph_len, prompt_id, scalar):
    n, d = x.shape
    e = edge_index.shape[1]
    b_graphs = graph_len.shape[0]
    n_per = n // b_graphs
    hdim = params[0][2].shape[1]

    SPREAD = 64
    grp = NUM_WORKERS * NBUF * CHUNK
    ep = -(-e // grp) * grp
    pad_ar = jnp.arange(ep - e, dtype=jnp.int32)
    srcp = jnp.concatenate([edge_index[0], pad_ar % n]).reshape(-1, CHUNK)
    dstp = jnp.concatenate([edge_index[1], n + pad_ar % SPREAD]).reshape(-1, CHUNK)
    spd = jnp.stack([srcp, dstp], axis=1)

    zeros_d = jnp.zeros((n + SPREAD, d), jnp.float32)
    zeros_h = jnp.zeros((n + SPREAD, hdim), jnp.float32)

    agg = _sc_scatter_sum(x, spd, zeros_d)
    w1, b1, w2, b2, gamma, beta = params[0]
    h, p1 = _tc_layer(x, agg, w0, w1, b1, w2, b2, gamma, beta, b_graphs, n_per)

    w1, b1, w2, b2, gamma, beta = params[1]
    agg = _sc_scatter_sum(h, spd, zeros_h)
    h, p2 = _tc_layer(h, agg, None, w1, b1, w2, b2, gamma, beta, b_graphs, n_per)

    w1, b1, w2, b2, gamma, beta = params[2]
    agg = _sc_scatter_sum(h, spd, zeros_h)
    h, p3 = _tc_layer(h, agg, None, w1, b1, w2, b2, gamma, beta, b_graphs, n_per)

    xcat, emb = _tc_combine(p1, p2, p3, n_per)
    return (xcat, emb)


# speedup vs baseline: 3.6138x; 1.2801x over previous
import functools

import jax
import jax.numpy as jnp
from jax import lax
from jax.experimental import pallas as pl
from jax.experimental.pallas import tpu as pltpu
from jax.experimental.pallas import tpu_sc as plsc

NUM_CORES = 2
NUM_SUBCORES = 16
NUM_WORKERS = NUM_CORES * NUM_SUBCORES
CHUNK = 128
NBUF = 3


def _sc_scatter_sum(h, spd, zeros_nf):
    n, f = h.shape
    nacc = zeros_nf.shape[0]
    cpw = spd.shape[0] // NUM_WORKERS

    mesh = plsc.VectorSubcoreMesh(
        core_axis_name="c", subcore_axis_name="s",
        num_cores=NUM_CORES, num_subcores=NUM_SUBCORES)

    @functools.partial(
        pl.kernel,
        out_type=jax.ShapeDtypeStruct((NUM_CORES, n, f), jnp.float32),
        mesh=mesh,
        scratch_types=[
            [pltpu.VMEM((2, 2, CHUNK), jnp.int32) for _ in range(NBUF)],
            [pltpu.VMEM((CHUNK, f), jnp.float32) for _ in range(NBUF)],
            pltpu.VMEM_SHARED((nacc, f), jnp.float32),
            [pltpu.SemaphoreType.DMA for _ in range(NBUF)],
            [pltpu.SemaphoreType.DMA for _ in range(NBUF)],
        ],
        compiler_params=pltpu.CompilerParams(use_tc_tiling_on_sc=False),
    )
    def k(h_hbm, spd_hbm, z_hbm, out_hbm, idx, rows, acc, gsem, isem):
        cid = lax.axis_index("c")
        sid = lax.axis_index("s")
        wid = cid * NUM_SUBCORES + sid
        base = wid * cpw

        def fire_idx(c, j, p):
            pltpu.async_copy(spd_hbm.at[c], idx[j].at[p], isem[j])

        def wait_idx(j, p):
            pltpu.make_async_copy(
                spd_hbm.at[base], idx[j].at[p], isem[j]).wait()

        def fire_gather(c, j, p):
            pltpu.async_copy(h_hbm.at[idx[j].at[p, 0]], rows[j], gsem[j])

        for j in range(NBUF):
            fire_idx(base + j, j, 0)
        for j in range(NBUF):
            wait_idx(j, 0)
            fire_gather(base + j, j, 0)
            fire_idx(base + j + NBUF, j, 1)

        rz = nacc // NUM_SUBCORES
        pltpu.sync_copy(z_hbm.at[pl.ds(sid * rz, rz)],
                        acc.at[pl.ds(sid * rz, rz)])
        plsc.subcore_barrier()

        def body(g, _):
            p = lax.rem(g, 2)
            for j in range(NBUF):
                c = g * NBUF + j
                pltpu.make_async_copy(
                    h_hbm.at[idx[j].at[p, 0]], rows[j], gsem[j]).wait()
                pltpu.sync_copy(rows[j], acc.at[idx[j].at[p, 1]], add=True)

                @pl.when(c + NBUF < cpw)
                def _():
                    wait_idx(j, 1 - p)
                    fire_gather(base + c + NBUF, j, 1 - p)

                    @pl.when(c + 2 * NBUF < cpw)
                    def _():
                        fire_idx(base + c + 2 * NBUF, j, p)
            return ()

        lax.fori_loop(0, cpw // NBUF, body, (), unroll=False)

        plsc.subcore_barrier()

        rw = n // NUM_SUBCORES
        pltpu.sync_copy(acc.at[pl.ds(sid * rw, rw)],
                        out_hbm.at[cid, pl.ds(sid * rw, rw)])

    return k(h, spd, zeros_nf)


def _tc_layer(h, agg, w0, w1, b1, w2, b2, gamma, beta, b_graphs, n_per):
    n, f = h.shape
    hdim = w2.shape[1]

    def body(h_ref, a_ref, w0_ref, w1_ref, b1_ref, w2_ref, b2_ref,
             g_ref, be_ref, out_ref, pool_ref):
        a = a_ref[...]
        rst = h_ref[...] + a[0] + a[1]
        if w0 is not None:
            rst = rst * w0_ref[...]
        z = jnp.dot(rst, w1_ref[...], preferred_element_type=jnp.float32)
        z = jnp.maximum(z + b1_ref[...], 0.0)
        z = jnp.dot(z, w2_ref[...], preferred_element_type=jnp.float32)
        hh = jnp.maximum(z + b2_ref[...], 0.0)
        mean = jnp.mean(hh, axis=0, keepdims=True)
        var = jnp.mean((hh - mean) * (hh - mean), axis=0, keepdims=True)
        hh = (hh - mean) * lax.rsqrt(var + 1e-5) * g_ref[...] + be_ref[...]
        out_ref[...] = hh
        rows = lax.broadcasted_iota(jnp.int32, (b_graphs, n), 1) // n_per
        seg = lax.broadcasted_iota(jnp.int32, (b_graphs, n), 0)
        ind = jnp.where(rows == seg, 1.0, 0.0).astype(jnp.float32)
        pool_ref[...] = jnp.dot(ind, hh, preferred_element_type=jnp.float32)

    args = [h, agg]
    if w0 is None:
        w0_in = jnp.zeros((1, f), jnp.float32)
    else:
        w0_in = w0
    args += [w0_in, w1, b1.reshape(1, -1), w2, b2.reshape(1, -1),
             gamma.reshape(1, -1), beta.reshape(1, -1)]
    return pl.pallas_call(
        body,
        out_shape=[
            jax.ShapeDtypeStruct((n, hdim), jnp.float32),
            jax.ShapeDtypeStruct((b_graphs, hdim), jnp.float32),
        ],
    )(*args)


def _tc_combine(p1, p2, p3, n_per):
    b, hdim = p1.shape

    def body(p1_ref, p2_ref, p3_ref, xcat_ref, emb_ref):
        xcat_ref[:, 0:hdim] = p1_ref[...]
        xcat_ref[:, hdim:2 * hdim] = p2_ref[...]
        xcat_ref[:, 2 * hdim:3 * hdim] = p3_ref[...]
        emb_ref[:, 0:hdim] = p1_ref[...] * (1.0 / n_per)
        emb_ref[:, hdim:2 * hdim] = p2_ref[...] * (1.0 / n_per)
        emb_ref[:, 2 * hdim:3 * hdim] = p3_ref[...] * (1.0 / n_per)

    return pl.pallas_call(
        body,
        out_shape=[
            jax.ShapeDtypeStruct((b, 3 * hdim), jnp.float32),
            jax.ShapeDtypeStruct((b, 3 * hdim), jnp.float32),
        ],
    )(p1, p2, p3)


def kernel(x, w0, params, edge_index, graph_len, prompt_id, scalar):
    n, d = x.shape
    e = edge_index.shape[1]
    b_graphs = graph_len.shape[0]
    n_per = n // b_graphs
    hdim = params[0][2].shape[1]

    SPREAD = 32
    grp = NUM_WORKERS * NBUF * CHUNK
    ep = -(-e // grp) * grp
    pad_ar = jnp.arange(ep - e, dtype=jnp.int32)
    srcp = jnp.concatenate([edge_index[0], pad_ar % n]).reshape(-1, CHUNK)
    dstp = jnp.concatenate([edge_index[1], n + pad_ar % SPREAD]).reshape(-1, CHUNK)
    spd = jnp.stack([srcp, dstp], axis=1)

    zeros_d = jnp.zeros((n + SPREAD, d), jnp.float32)
    zeros_h = jnp.zeros((n + SPREAD, hdim), jnp.float32)

    agg = _sc_scatter_sum(x, spd, zeros_d)
    w1, b1, w2, b2, gamma, beta = params[0]
    h, p1 = _tc_layer(x, agg, w0, w1, b1, w2, b2, gamma, beta, b_graphs, n_per)

    w1, b1, w2, b2, gamma, beta = params[1]
    agg = _sc_scatter_sum(h, spd, zeros_h)
    h, p2 = _tc_layer(h, agg, None, w1, b1, w2, b2, gamma, beta, b_graphs, n_per)

    w1, b1, w2, b2, gamma, beta = params[2]
    agg = _sc_scatter_sum(h, spd, zeros_h)
    h, p3 = _tc_layer(h, agg, None, w1, b1, w2, b2, gamma, beta, b_graphs, n_per)

    xcat, emb = _tc_combine(p1, p2, p3, n_per)
    return (xcat, emb)
